# baseline (speedup 1.0000x reference)
import jax
import jax.numpy as jnp
from jax.experimental import pallas as pl
from jax.experimental.pallas import tpu as pltpu

_K = 512
_D = 32
_KC = 64


def _vq_kernel(z_ref, emb_ref, out_ref, minv_ref, mini_ref):
    c = pl.program_id(1)
    nc = pl.num_programs(1)
    zb = z_ref[0]
    acc = jnp.zeros((_KC, zb.shape[1]), jnp.float32)
    for d in range(_D):
        t = zb[d : d + 1, :] - emb_ref[:, d : d + 1]
        acc = acc + t * t
    m = jnp.min(acc, axis=0, keepdims=True)
    iota = jax.lax.broadcasted_iota(jnp.int32, acc.shape, 0)
    loc = jnp.min(jnp.where(acc == m, iota, _K), axis=0, keepdims=True) + c * _KC

    @pl.when(c == 0)
    def _init():
        minv_ref[...] = m
        mini_ref[...] = loc

    @pl.when(c != 0)
    def _update():
        upd = m < minv_ref[...]
        mini_ref[...] = jnp.where(upd, loc, mini_ref[...])
        minv_ref[...] = jnp.where(upd, m, minv_ref[...])

    @pl.when(c == nc - 1)
    def _emit():
        out_ref[0] = mini_ref[...]


def kernel(z_e_x, emb):
    B, D, H, W = z_e_x.shape
    HW = H * W
    z3 = z_e_x.reshape(B, D, HW)
    out = pl.pallas_call(
        _vq_kernel,
        grid=(B, _K // _KC),
        in_specs=[
            pl.BlockSpec((1, D, HW), lambda b, c: (b, 0, 0)),
            pl.BlockSpec((_KC, D), lambda b, c: (c, 0)),
        ],
        out_specs=pl.BlockSpec((1, 1, HW), lambda b, c: (b, 0, 0)),
        out_shape=jax.ShapeDtypeStruct((B, 1, HW), jnp.int32),
        scratch_shapes=[
            pltpu.VMEM((1, HW), jnp.float32),
            pltpu.VMEM((1, HW), jnp.int32),
        ],
        compiler_params=pltpu.CompilerParams(
            dimension_semantics=("arbitrary", "arbitrary"),
        ),
    )(z3, emb)
    return out.reshape(B, H, W)


# speedup vs baseline: 2.0356x; 2.0356x over previous
import jax
import jax.numpy as jnp
from jax.experimental import pallas as pl
from jax.experimental.pallas import tpu as pltpu

_K = 512
_D = 32
_SL = 32
_LN = 128


def _vq_kernel(z_ref, emb_ref, out_ref):
    KU = 8

    def body(i, carry):
        minv, mini = carry
        k0 = i * KU
        erows = emb_ref[pl.ds(k0, KU), :]
        accs = [jnp.zeros((_SL, _LN), jnp.float32) for _ in range(KU)]
        for d in range(_D):
            zd = z_ref[d]
            for j in range(KU):
                t = zd - erows[j : j + 1, d : d + 1]
                accs[j] = accs[j] + t * t
        for j in range(KU):
            upd = accs[j] < minv
            minv = jnp.where(upd, accs[j], minv)
            mini = jnp.where(upd, k0 + j, mini)
        return minv, mini

    init = (jnp.full((_SL, _LN), jnp.inf, jnp.float32),
            jnp.zeros((_SL, _LN), jnp.int32))
    _, mini = jax.lax.fori_loop(0, _K // KU, body, init)
    out_ref[...] = mini


def kernel(z_e_x, emb):
    B, D, H, W = z_e_x.shape
    zt = jnp.transpose(z_e_x, (1, 0, 2, 3)).reshape(D, _SL, _LN)
    out = pl.pallas_call(
        _vq_kernel,
        in_specs=[
            pl.BlockSpec((D, _SL, _LN), lambda: (0, 0, 0)),
            pl.BlockSpec((_K, D), lambda: (0, 0)),
        ],
        out_specs=pl.BlockSpec((_SL, _LN), lambda: (0, 0)),
        out_shape=jax.ShapeDtypeStruct((_SL, _LN), jnp.int32),
    )(zt, emb)
    return out.reshape(B, H, W)


# speedup vs baseline: 2.2023x; 1.0819x over previous
import jax
import jax.numpy as jnp
from jax.experimental import pallas as pl
from jax.experimental.pallas import tpu as pltpu

_K = 512
_D = 32
_SL = 32
_LN = 128


def _vq_kernel(z_ref, emb_ref, out_ref):
    KU = 8

    def body(i, carry):
        minv, mini = carry
        k0 = i * KU
        accs = [jnp.zeros((_SL, _LN), jnp.float32) for _ in range(KU)]
        for d in range(_D):
            zd = z_ref[d]
            for j in range(KU):
                t = zd - emb_ref[k0 + j, d]
                accs[j] = accs[j] + t * t
        for j in range(KU):
            upd = accs[j] < minv
            minv = jnp.where(upd, accs[j], minv)
            mini = jnp.where(upd, k0 + j, mini)
        return minv, mini

    init = (jnp.full((_SL, _LN), jnp.inf, jnp.float32),
            jnp.zeros((_SL, _LN), jnp.int32))
    _, mini = jax.lax.fori_loop(0, _K // KU, body, init)
    out_ref[...] = mini


def kernel(z_e_x, emb):
    B, D, H, W = z_e_x.shape
    zt = jnp.transpose(z_e_x, (1, 0, 2, 3)).reshape(D, _SL, _LN)
    out = pl.pallas_call(
        _vq_kernel,
        in_specs=[
            pl.BlockSpec((D, _SL, _LN), lambda: (0, 0, 0)),
            pl.BlockSpec(memory_space=pltpu.SMEM),
        ],
        out_specs=pl.BlockSpec((_SL, _LN), lambda: (0, 0)),
        out_shape=jax.ShapeDtypeStruct((_SL, _LN), jnp.int32),
    )(zt, emb)
    return out.reshape(B, H, W)
